# Initial kernel scaffold; baseline (speedup 1.0000x reference)
import functools

import jax
import jax.numpy as jnp
from jax import lax
from jax.experimental import pallas as pl
from jax.experimental.pallas import tpu as pltpu
from jax.experimental.pallas import tpu_sc as plsc

N = 10000
NP = 10016
E = 320000
C = 128
NC, NS = 2, 16
NW = NC * NS
GA = 79
RA = NW * GA
EP = RA * C
GD = 2 * GA
RPT = NP // NS

_mesh = plsc.VectorSubcoreMesh(core_axis_name="c", subcore_axis_name="s")


def _deg_body(idx_hbm, ones_hbm, zeros_hbm, out_hbm, idx_v, ones_v, sem, acc):
    c = lax.axis_index("c")
    s = lax.axis_index("s")
    pltpu.sync_copy(zeros_hbm.at[pl.ds(s * RPT, RPT)], acc.at[pl.ds(s * RPT, RPT)])
    pltpu.sync_copy(ones_hbm, ones_v)
    pltpu.sync_copy(idx_hbm.at[c, pl.ds(s * GD, GD)], idx_v)
    plsc.subcore_barrier()
    def fire(g, x):
        pltpu.async_copy(ones_v, acc.at[idx_v.at[g]], sem, add=True)
        return x
    lax.fori_loop(0, GD, fire, 0)
    def drain(g, x):
        pltpu.make_async_copy(ones_v, acc.at[idx_v.at[g]], sem).wait()
        return x
    lax.fori_loop(0, GD, drain, 0)
    plsc.subcore_barrier()
    pltpu.sync_copy(acc.at[pl.ds(s * RPT, RPT)], out_hbm.at[c, pl.ds(s * RPT, RPT)])


_deg_call = pl.kernel(
    _deg_body,
    out_type=jax.ShapeDtypeStruct((NC, NP, 4), jnp.float32),
    mesh=_mesh,
    scratch_types=[
        pltpu.VMEM((GD, C), jnp.int32),
        pltpu.VMEM((C, 4), jnp.float32),
        pltpu.SemaphoreType.DMA,
        pltpu.MemorySpace.VMEM_SHARED((NP, 4), jnp.float32),
    ],
)


def _agg_body(table_hbm, src_hbm, dst_hbm, zeros_hbm, out_hbm,
              src_v, dst_v, msg_v, gsem, ssem, acc):
    c = lax.axis_index("c")
    s = lax.axis_index("s")
    w = c * NS + s
    pltpu.sync_copy(zeros_hbm.at[pl.ds(s * RPT, RPT)], acc.at[pl.ds(s * RPT, RPT)])
    pltpu.sync_copy(src_hbm.at[pl.ds(w * GA, GA)], src_v)
    pltpu.sync_copy(dst_hbm.at[pl.ds(w * GA, GA)], dst_v)
    def gfire(g, x):
        pltpu.async_copy(table_hbm.at[src_v.at[g]], msg_v.at[g], gsem)
        return x
    lax.fori_loop(0, GA, gfire, 0)
    def gdrain(g, x):
        pltpu.make_async_copy(table_hbm.at[src_v.at[g]], msg_v.at[g], gsem).wait()
        return x
    lax.fori_loop(0, GA, gdrain, 0)
    plsc.subcore_barrier()
    def sfire(g, x):
        pltpu.async_copy(msg_v.at[g], acc.at[dst_v.at[g]], ssem, add=True)
        return x
    lax.fori_loop(0, GA, sfire, 0)
    def sdrain(g, x):
        pltpu.make_async_copy(msg_v.at[g], acc.at[dst_v.at[g]], ssem).wait()
        return x
    lax.fori_loop(0, GA, sdrain, 0)
    plsc.subcore_barrier()
    pltpu.sync_copy(acc.at[pl.ds(s * RPT, RPT)], out_hbm.at[c, pl.ds(s * RPT, RPT)])


_agg_call = pl.kernel(
    _agg_body,
    out_type=jax.ShapeDtypeStruct((NC, NP, 4), jnp.float32),
    mesh=_mesh,
    scratch_types=[
        pltpu.VMEM((GA, C), jnp.int32),
        pltpu.VMEM((GA, C), jnp.int32),
        pltpu.VMEM((GA, C, 4), jnp.float32),
        pltpu.SemaphoreType.DMA,
        pltpu.SemaphoreType.DMA,
        pltpu.MemorySpace.VMEM_SHARED((NP, 4), jnp.float32),
    ],
)


def _tc1_body(x_ref, w1_ref, deg_ref, msg_ref, ns_ref, nd_ref):
    h = jnp.dot(x_ref[...], w1_ref[...], preferred_element_type=jnp.float32)
    od = deg_ref[0][:, 0:1]
    idg = deg_ref[1][:, 0:1]
    ns = jnp.where(od > 0, lax.rsqrt(jnp.maximum(od, 1.0)), 0.0)
    nd = jnp.where(idg > 0, lax.rsqrt(jnp.maximum(idg, 1.0)), 0.0)
    msg_ref[...] = h * ns
    ns_ref[...] = ns
    nd_ref[...] = nd


_tc1_call = pl.pallas_call(
    _tc1_body,
    out_shape=(
        jax.ShapeDtypeStruct((NP, 4), jnp.float32),
        jax.ShapeDtypeStruct((NP, 1), jnp.float32),
        jax.ShapeDtypeStruct((NP, 1), jnp.float32),
    ),
)


def _tc2_body(p_ref, ns_ref, nd_ref, b1_ref, msg2_ref):
    t = (p_ref[0] + p_ref[1]) * nd_ref[...] + b1_ref[...]
    msg2_ref[...] = jnp.maximum(t, 0.0) * ns_ref[...]


_tc2_call = pl.pallas_call(
    _tc2_body,
    out_shape=jax.ShapeDtypeStruct((NP, 4), jnp.float32),
)


def _tc3_body(p_ref, nd_ref, w2_ref, b2_ref, out_ref):
    t = (p_ref[0] + p_ref[1]) * nd_ref[...]
    out_ref[...] = (
        jnp.dot(t, w2_ref[...], preferred_element_type=jnp.float32) + b2_ref[...]
    )


_tc3_call = pl.pallas_call(
    _tc3_body,
    out_shape=jax.ShapeDtypeStruct((NP, 40), jnp.float32),
)


def kernel(in_feat, edge_index, W1, b1, W2, b2):
    src = edge_index[0].astype(jnp.int32)
    dst = edge_index[1].astype(jnp.int32)
    padv = jnp.full((EP - E,), N, jnp.int32)
    srcp = jnp.concatenate([src, padv]).reshape(RA, C)
    dstp = jnp.concatenate([dst, padv]).reshape(RA, C)
    deg_idx = jnp.stack([srcp, dstp])
    ones = jnp.ones((C, 4), jnp.float32)
    zeros = jnp.zeros((NP, 4), jnp.float32)
    xp = jnp.pad(in_feat, ((0, NP - N), (0, 0)))

    deg = _deg_call(deg_idx, ones, zeros)
    msg1, ns, nd = _tc1_call(xp, W1, deg)
    p1 = _agg_call(msg1, srcp, dstp, zeros)
    msg2 = _tc2_call(p1, ns, nd, b1.reshape(1, 4))
    p2 = _agg_call(msg2, srcp, dstp, zeros)
    out = _tc3_call(p2, nd, W2, b2.reshape(1, 40))
    return out[:N]


# speedup vs baseline: 17.6610x; 17.6610x over previous
import jax
import jax.numpy as jnp
from jax import lax
from jax.experimental import pallas as pl
from jax.experimental.pallas import tpu as pltpu
from jax.experimental.pallas import tpu_sc as plsc

N = 10000
NP = 10112
NP2 = 2 * NP
E = 320000
C = 128
NC, NS = 2, 16
NW = NC * NS
GA = 80
RA = NW * GA
EP = RA * C
GD = 2 * GA
RD = NW * GD
DPR = 40
APR = 20

_mesh = plsc.VectorSubcoreMesh(core_axis_name="c", subcore_axis_name="s")
_sc_params = pltpu.CompilerParams(
    use_tc_tiling_on_sc=False, needs_layout_passes=False)


def _deg_body(idx_hbm, out_hbm, idx_v, acc_v):
    c = lax.axis_index("c")
    s = lax.axis_index("s")
    w = c * NS + s
    zero16 = jnp.zeros((16,), jnp.float32)
    one16 = jnp.ones((16,), jnp.float32)

    def zero(i, x):
        acc_v[pl.ds(i * 16, 16)] = zero16
        return x
    lax.fori_loop(0, NP2 // 16, zero, 0)

    def dpass(p, x):
        pltpu.sync_copy(idx_hbm.at[pl.ds(w * GD + p * DPR, DPR)], idx_v)

        def count(g, y):
            for j in range(C // 16):
                idx16 = idx_v[g, pl.ds(j * 16, 16)]
                plsc.addupdate_scatter(acc_v, [idx16], one16)
            return y
        lax.fori_loop(0, DPR, count, 0)
        return x
    lax.fori_loop(0, GD // DPR, dpass, 0)
    pltpu.sync_copy(acc_v, out_hbm.at[w])


_deg_call = pl.kernel(
    _deg_body,
    out_type=jax.ShapeDtypeStruct((NW, NP2), jnp.float32),
    mesh=_mesh,
    scratch_types=[
        pltpu.VMEM((DPR, C), jnp.int32),
        pltpu.VMEM((NP2,), jnp.float32),
    ],
    compiler_params=_sc_params,
)


def _agg_body(table_hbm, src_hbm, dst_hbm, out_hbm, table_v, src_v, dst_v, acc_v):
    c = lax.axis_index("c")
    s = lax.axis_index("s")
    w = c * NS + s
    zero16 = jnp.zeros((16,), jnp.float32)

    def zero(i, x):
        for f in range(4):
            acc_v[f, pl.ds(i * 16, 16)] = zero16
        return x
    lax.fori_loop(0, NP // 16, zero, 0)
    pltpu.sync_copy(table_hbm, table_v)

    feat = [jnp.full((16,), f, jnp.int32) for f in range(4)]

    def apass(p, x):
        pltpu.sync_copy(src_hbm.at[pl.ds(w * GA + p * APR, APR)], src_v)
        pltpu.sync_copy(dst_hbm.at[pl.ds(w * GA + p * APR, APR)], dst_v)

        def agg(g, y):
            for j in range(C // 16):
                s16 = src_v[g, pl.ds(j * 16, 16)]
                d16 = dst_v[g, pl.ds(j * 16, 16)]
                for f in range(4):
                    v = plsc.load_gather(table_v, [feat[f], s16])
                    plsc.addupdate_scatter(acc_v, [feat[f], d16], v)
            return y
        lax.fori_loop(0, APR, agg, 0)
        return x
    lax.fori_loop(0, GA // APR, apass, 0)
    pltpu.sync_copy(acc_v, out_hbm.at[w])


_agg_call = pl.kernel(
    _agg_body,
    out_type=jax.ShapeDtypeStruct((NW, 4, NP), jnp.float32),
    mesh=_mesh,
    scratch_types=[
        pltpu.VMEM((4, NP), jnp.float32),
        pltpu.VMEM((APR, C), jnp.int32),
        pltpu.VMEM((APR, C), jnp.int32),
        pltpu.VMEM((4, NP), jnp.float32),
    ],
    compiler_params=_sc_params,
)


def _tc1_body(xt_ref, w1t_ref, degp_ref, msg_ref, ns_ref, nd_ref):
    deg = jnp.sum(degp_ref[...], axis=0)
    od = deg[0:1]
    idg = deg[1:2]
    ns = jnp.where(od > 0, lax.rsqrt(jnp.maximum(od, 1.0)), 0.0)
    nd = jnp.where(idg > 0, lax.rsqrt(jnp.maximum(idg, 1.0)), 0.0)
    h = jnp.dot(w1t_ref[...], xt_ref[...], preferred_element_type=jnp.float32)
    msg_ref[...] = h * ns
    ns_ref[...] = ns
    nd_ref[...] = nd


_tc1_call = pl.pallas_call(
    _tc1_body,
    out_shape=(
        jax.ShapeDtypeStruct((4, NP), jnp.float32),
        jax.ShapeDtypeStruct((1, NP), jnp.float32),
        jax.ShapeDtypeStruct((1, NP), jnp.float32),
    ),
)


def _tc2_body(p_ref, ns_ref, nd_ref, b1_ref, msg2_ref):
    agg = jnp.sum(p_ref[...], axis=0)
    t = agg * nd_ref[...] + b1_ref[...]
    msg2_ref[...] = jnp.maximum(t, 0.0) * ns_ref[...]


_tc2_call = pl.pallas_call(
    _tc2_body,
    out_shape=jax.ShapeDtypeStruct((4, NP), jnp.float32),
)


def _tc3_body(p_ref, nd_ref, w2t_ref, b2_ref, out_ref):
    agg = jnp.sum(p_ref[...], axis=0)
    t = agg * nd_ref[...]
    out_ref[...] = (
        jnp.dot(w2t_ref[...], t, preferred_element_type=jnp.float32) + b2_ref[...]
    )


_tc3_call = pl.pallas_call(
    _tc3_body,
    out_shape=jax.ShapeDtypeStruct((40, NP), jnp.float32),
)


def kernel(in_feat, edge_index, W1, b1, W2, b2):
    src = edge_index[0].astype(jnp.int32)
    dst = edge_index[1].astype(jnp.int32)
    padv = jnp.full((EP - E,), N, jnp.int32)
    srcp = jnp.concatenate([src, padv]).reshape(RA, C)
    dstp = jnp.concatenate([dst, padv]).reshape(RA, C)
    deg_idx = jnp.concatenate([srcp, dstp + NP]).reshape(RD, C)
    xt = jnp.pad(in_feat, ((0, NP - N), (0, 0))).T

    degp = _deg_call(deg_idx).reshape(NW, 2, NP)
    msg1, ns, nd = _tc1_call(xt, W1.T, degp)
    p1 = _agg_call(msg1, srcp, dstp)
    msg2 = _tc2_call(p1, ns, nd, b1.reshape(4, 1))
    p2 = _agg_call(msg2, srcp, dstp)
    out = _tc3_call(p2, nd, W2.T, b2.reshape(40, 1))
    return out[:, :N].T


# speedup vs baseline: 21.6449x; 1.2256x over previous
import jax
import jax.numpy as jnp
from jax import lax
from jax.experimental import pallas as pl
from jax.experimental.pallas import tpu as pltpu
from jax.experimental.pallas import tpu_sc as plsc

N = 10000
NP = 10112
NP2 = 2 * NP
E = 320000
C = 128
NC, NS = 2, 16
NW = NC * NS
GA = 80
GA0 = 112
GA1 = 48
RA = NW * GA
EP = RA * C
GD = 2 * GA
RD = NW * GD
DPR = 40
APR = 16
TREP = 4

_mesh = plsc.VectorSubcoreMesh(core_axis_name="c", subcore_axis_name="s")
_sc_params = pltpu.CompilerParams(
    use_tc_tiling_on_sc=False, needs_layout_passes=False)


def _deg_body(idx_hbm, out_hbm, idx_v, acc_v):
    c = lax.axis_index("c")
    s = lax.axis_index("s")
    w = c * NS + s
    zero16 = jnp.zeros((16,), jnp.float32)
    one16 = jnp.ones((16,), jnp.float32)

    def zero(i, x):
        acc_v[pl.ds(i * 16, 16)] = zero16
        return x
    lax.fori_loop(0, NP2 // 16, zero, 0)

    def dpass(p, x):
        pltpu.sync_copy(idx_hbm.at[pl.ds(w * GD + p * DPR, DPR)], idx_v)

        @plsc.parallel_loop(0, DPR, unroll=4)
        def count(g):
            for j in range(C // 16):
                idx16 = idx_v[g, pl.ds(j * 16, 16)]
                plsc.addupdate_scatter(acc_v, [idx16], one16)
        return x
    lax.fori_loop(0, GD // DPR, dpass, 0)
    pltpu.sync_copy(acc_v, out_hbm.at[w])


_deg_call = pl.kernel(
    _deg_body,
    out_type=jax.ShapeDtypeStruct((NW, NP2), jnp.float32),
    mesh=_mesh,
    scratch_types=[
        pltpu.VMEM((DPR, C), jnp.int32),
        pltpu.VMEM((NP2,), jnp.float32),
    ],
    compiler_params=_sc_params,
)


def _agg_body(table_hbm, src_hbm, dst_hbm, out_hbm, table_v, src_v, dst_v, acc_v):
    c = lax.axis_index("c")
    s = lax.axis_index("s")
    w = c * NS + s
    zero16 = jnp.zeros((16,), jnp.float32)

    def zero(i, x):
        for f in range(4):
            acc_v[f, pl.ds(i * 16, 16)] = zero16
        return x
    lax.fori_loop(0, NP // 16, zero, 0)
    pltpu.sync_copy(table_hbm.at[c * 2 + (s % 2)], table_v)

    feat = [jnp.full((16,), f, jnp.int32) for f in range(4)]
    my_rows = jnp.where(c == 0, GA0, GA1)
    row0 = jnp.where(c == 0, s * GA0, NS * GA0 + s * GA1)

    def apass(p, x):
        pltpu.sync_copy(src_hbm.at[pl.ds(row0 + p * APR, APR)], src_v)
        pltpu.sync_copy(dst_hbm.at[pl.ds(row0 + p * APR, APR)], dst_v)

        @plsc.parallel_loop(0, APR, unroll=4)
        def agg(g):
            for j in range(C // 16):
                s16 = src_v[g, pl.ds(j * 16, 16)]
                d16 = dst_v[g, pl.ds(j * 16, 16)]
                for f in range(4):
                    v = plsc.load_gather(table_v, [feat[f], s16])
                    plsc.addupdate_scatter(acc_v, [feat[f], d16], v)
        return x
    lax.fori_loop(0, my_rows // APR, apass, 0)
    pltpu.sync_copy(acc_v, out_hbm.at[w])


_agg_call = pl.kernel(
    _agg_body,
    out_type=jax.ShapeDtypeStruct((NW, 4, NP), jnp.float32),
    mesh=_mesh,
    scratch_types=[
        pltpu.VMEM((4, NP), jnp.float32),
        pltpu.VMEM((APR, C), jnp.int32),
        pltpu.VMEM((APR, C), jnp.int32),
        pltpu.VMEM((4, NP), jnp.float32),
    ],
    compiler_params=_sc_params,
)


def _tc1_body(xt_ref, w1t_ref, degp_ref, msg_ref, ns_ref, nd_ref):
    deg = jnp.sum(degp_ref[...], axis=0)
    od = deg[0:1]
    idg = deg[1:2]
    ns = jnp.where(od > 0, lax.rsqrt(jnp.maximum(od, 1.0)), 0.0)
    nd = jnp.where(idg > 0, lax.rsqrt(jnp.maximum(idg, 1.0)), 0.0)
    h = jnp.dot(w1t_ref[...], xt_ref[...], preferred_element_type=jnp.float32)
    msg_ref[...] = jnp.broadcast_to((h * ns)[None], (TREP, 4, NP))
    ns_ref[...] = ns
    nd_ref[...] = nd


_tc1_call = pl.pallas_call(
    _tc1_body,
    out_shape=(
        jax.ShapeDtypeStruct((TREP, 4, NP), jnp.float32),
        jax.ShapeDtypeStruct((1, NP), jnp.float32),
        jax.ShapeDtypeStruct((1, NP), jnp.float32),
    ),
)


def _tc2_body(p_ref, ns_ref, nd_ref, b1_ref, msg2_ref):
    agg = jnp.sum(p_ref[...], axis=0)
    t = agg * nd_ref[...] + b1_ref[...]
    m = jnp.maximum(t, 0.0) * ns_ref[...]
    msg2_ref[...] = jnp.broadcast_to(m[None], (TREP, 4, NP))


_tc2_call = pl.pallas_call(
    _tc2_body,
    out_shape=jax.ShapeDtypeStruct((TREP, 4, NP), jnp.float32),
)


def _tc3_body(p_ref, nd_ref, w2t_ref, b2_ref, out_ref):
    agg = jnp.sum(p_ref[...], axis=0)
    t = agg * nd_ref[...]
    out_ref[...] = (
        jnp.dot(w2t_ref[...], t, preferred_element_type=jnp.float32) + b2_ref[...]
    )


_tc3_call = pl.pallas_call(
    _tc3_body,
    out_shape=jax.ShapeDtypeStruct((40, NP), jnp.float32),
)


def kernel(in_feat, edge_index, W1, b1, W2, b2):
    src = edge_index[0].astype(jnp.int32)
    dst = edge_index[1].astype(jnp.int32)
    padv = jnp.full((EP - E,), N, jnp.int32)
    srcp = jnp.concatenate([src, padv]).reshape(RA, C)
    dstp = jnp.concatenate([dst, padv]).reshape(RA, C)
    deg_idx = jnp.concatenate([srcp, dstp + NP]).reshape(RD, C)
    xt = jnp.pad(in_feat, ((0, NP - N), (0, 0))).T

    degp = _deg_call(deg_idx).reshape(NW, 2, NP)
    msg1, ns, nd = _tc1_call(xt, W1.T, degp)
    p1 = _agg_call(msg1, srcp, dstp)
    msg2 = _tc2_call(p1, ns, nd, b1.reshape(4, 1))
    p2 = _agg_call(msg2, srcp, dstp)
    out = _tc3_call(p2, nd, W2.T, b2.reshape(40, 1))
    return out[:, :N].T
